# baseline (speedup 1.0000x reference)
import functools

import jax
import jax.numpy as jnp
from jax import lax
from jax.experimental import pallas as pl
from jax.experimental.pallas import tpu as pltpu
from jax.experimental.pallas import tpu_sc as plsc

N = 10000
NP = 10240
E = 320000
GARBAGE = N

K = 128
CA = 80
CB = 160
ROWS_PER_SUB = NP // 16

_mesh = plsc.VectorSubcoreMesh(core_axis_name="c", subcore_axis_name="s")


G = 20
NCHUNK = 32 * CA


def _agg_pipeline(table_hbm, stage_fn, sd_g, rows2, acc_sh, sem_g, sem_s,
                  n_groups):
    def group(g, _):
        stage_fn(g)
        gathers = [None] * G
        scatters = [None] * G
        gathers[0] = pltpu.async_copy(table_hbm.at[sd_g.at[0, 0]],
                                      rows2.at[0], sem_g)
        for j in range(G):
            b = j & 1
            gathers[j].wait()
            if j + 1 < G:
                if j >= 1:
                    scatters[j - 1].wait()
                gathers[j + 1] = pltpu.async_copy(
                    table_hbm.at[sd_g.at[j + 1, 0]], rows2.at[1 - b], sem_g)
            scatters[j] = pltpu.async_copy(
                rows2.at[b], acc_sh.at[sd_g.at[j, 1]], sem_s, add=True)
        scatters[G - 2].wait()
        scatters[G - 1].wait()
        return 0

    lax.fori_loop(0, n_groups, group, 0)


@functools.partial(
    pl.kernel,
    out_type=[
        jax.ShapeDtypeStruct((2, NP, 128), jnp.float32),
        jax.ShapeDtypeStruct((2, NP, 128), jnp.float32),
    ],
    mesh=_mesh,
    scratch_types=[
        pltpu.VMEM((2 * G, 2, K), jnp.int32),
        pltpu.VMEM((2, K, 128), jnp.float32),
        pltpu.VMEM_SHARED((NP, 128), jnp.float32),
        pltpu.SemaphoreType.DMA,
        pltpu.SemaphoreType.DMA,
    ],
)
def _sc_agg0(x_hbm, sd_hbm, z128_hbm, ones_hbm,
             agg_out, deg_out,
             sd_g2, rows2, acc_sh, sem_g, sem_s):
    sd_g = sd_g2.at[pl.ds(0, G)]
    c = lax.axis_index("c")
    s = lax.axis_index("s")
    wid = c * 16 + s
    r0 = s * ROWS_PER_SUB
    pltpu.sync_copy(z128_hbm.at[pl.ds(r0, ROWS_PER_SUB)],
                    acc_sh.at[pl.ds(r0, ROWS_PER_SUB)])
    pltpu.sync_copy(ones_hbm, rows2.at[0])
    plsc.subcore_barrier()

    def deg_group(g, _):
        pltpu.sync_copy(sd_hbm.at[pl.ds(wid * CA + g * (2 * G), 2 * G)], sd_g2)
        scatters = [
            pltpu.async_copy(rows2.at[0], acc_sh.at[sd_g2.at[j, 1]], sem_s,
                             add=True)
            for j in range(2 * G)
        ]
        for sc in scatters:
            sc.wait()
        return 0

    lax.fori_loop(0, CA // (2 * G), deg_group, 0)
    plsc.subcore_barrier()
    pltpu.sync_copy(acc_sh.at[pl.ds(r0, ROWS_PER_SUB)],
                    deg_out.at[c, pl.ds(r0, ROWS_PER_SUB)])
    plsc.subcore_barrier()
    pltpu.sync_copy(z128_hbm.at[pl.ds(r0, ROWS_PER_SUB)],
                    acc_sh.at[pl.ds(r0, ROWS_PER_SUB)])
    plsc.subcore_barrier()
    def stage(g):
        pltpu.sync_copy(sd_hbm.at[pl.ds(wid * CA + g * G, G)], sd_g)

    _agg_pipeline(x_hbm, stage, sd_g, rows2, acc_sh, sem_g, sem_s, CA // G)
    plsc.subcore_barrier()
    pltpu.sync_copy(acc_sh.at[pl.ds(r0, ROWS_PER_SUB)],
                    agg_out.at[c, pl.ds(r0, ROWS_PER_SUB)])


@functools.partial(
    pl.kernel,
    out_type=jax.ShapeDtypeStruct((2, NP, 128), jnp.float32),
    mesh=_mesh,
    scratch_types=[
        pltpu.VMEM((G, 2, K), jnp.int32),
        pltpu.VMEM((2, K, 128), jnp.float32),
        pltpu.VMEM_SHARED((NP, 128), jnp.float32),
        pltpu.SemaphoreType.DMA,
        pltpu.SemaphoreType.DMA,
    ],
)
def _sc_agg1(h_hbm, sd_hbm, z128_hbm,
             agg_out,
             sd_g, rows2, acc_sh, sem_g, sem_s):
    c = lax.axis_index("c")
    s = lax.axis_index("s")
    r0 = s * ROWS_PER_SUB
    pltpu.sync_copy(z128_hbm.at[pl.ds(r0, ROWS_PER_SUB)],
                    acc_sh.at[pl.ds(r0, ROWS_PER_SUB)])
    plsc.subcore_barrier()
    off = c * NP

    def stage(g):
        pltpu.sync_copy(sd_hbm.at[pl.ds(s * CB + g * G, G)], sd_g)
        def add_off(j, _):
            def lane(i, _):
                v = sd_g[j, 0, pl.ds(i * 16, 16)]
                sd_g[j, 0, pl.ds(i * 16, 16)] = v + off
                return 0
            lax.fori_loop(0, K // 16, lane, 0)
            return 0
        lax.fori_loop(0, G, add_off, 0)

    _agg_pipeline(h_hbm, stage, sd_g, rows2, acc_sh, sem_g, sem_s, CB // G)
    plsc.subcore_barrier()
    pltpu.sync_copy(acc_sh.at[pl.ds(r0, ROWS_PER_SUB)],
                    agg_out.at[c, pl.ds(r0, ROWS_PER_SUB)])


def _tc1_body(aggp, degp, x, wl, bl, wr, out):
    a = aggp[0] + aggp[1]
    deg = degp[0, :, 0] + degp[1, :, 0]
    r = 1.0 / jnp.maximum(deg, 1.0)
    h = (jnp.dot(a * r[:, None], wl[:], preferred_element_type=jnp.float32)
         + bl[:] + jnp.dot(x[:], wr[:], preferred_element_type=jnp.float32))
    h = jnp.maximum(h, 0.0)
    out[0] = h[:, :128]
    out[1] = h[:, 128:]


def _tc2_body(aggh, degp, h0st, wl, bl, wr, wf0, bf0, wf1, bf1, wp, bp, out):
    agg = jnp.concatenate([aggh[0], aggh[1]], axis=1)
    h0 = jnp.concatenate([h0st[0], h0st[1]], axis=1)
    deg = degp[0, :, 0] + degp[1, :, 0]
    r = 1.0 / jnp.maximum(deg, 1.0)
    h = (jnp.dot(agg * r[:, None], wl[:], preferred_element_type=jnp.float32)
         + bl[:] + jnp.dot(h0, wr[:], preferred_element_type=jnp.float32))
    h = jnp.maximum(h, 0.0)
    h = jnp.maximum(jnp.dot(h, wf0[:], preferred_element_type=jnp.float32) + bf0[:], 0.0)
    h = jnp.maximum(jnp.dot(h, wf1[:], preferred_element_type=jnp.float32) + bf1[:], 0.0)
    out[:] = jnp.dot(h, wp[:], preferred_element_type=jnp.float32) + bp[:]


_R = 512
_GRID = NP // _R


def _full(shape):
    return pl.BlockSpec(shape, lambda i: tuple(0 for _ in shape))


_tc1 = pl.pallas_call(
    _tc1_body,
    grid=(_GRID,),
    in_specs=[
        pl.BlockSpec((2, _R, 128), lambda i: (0, i, 0)),
        pl.BlockSpec((2, _R, 128), lambda i: (0, i, 0)),
        pl.BlockSpec((_R, 128), lambda i: (i, 0)),
        _full((128, 256)),
        _full((1, 256)),
        _full((128, 256)),
    ],
    out_specs=pl.BlockSpec((2, _R, 128), lambda i: (0, i, 0)),
    out_shape=jax.ShapeDtypeStruct((2, NP, 128), jnp.float32),
)

_tc2 = pl.pallas_call(
    _tc2_body,
    grid=(_GRID,),
    in_specs=[
        pl.BlockSpec((2, _R, 128), lambda i: (0, i, 0)),
        pl.BlockSpec((2, _R, 128), lambda i: (0, i, 0)),
        pl.BlockSpec((2, _R, 128), lambda i: (0, i, 0)),
        _full((256, 256)),
        _full((1, 256)),
        _full((256, 256)),
        _full((256, 256)),
        _full((1, 256)),
        _full((256, 256)),
        _full((1, 256)),
        _full((256, 128)),
        _full((1, 128)),
    ],
    out_specs=pl.BlockSpec((_R, 128), lambda i: (i, 0)),
    out_shape=jax.ShapeDtypeStruct((NP, 128), jnp.float32),
)


def kernel(x, edge_index, Wl0, bl0, Wr0, Wl1, bl1, Wr1, Wf0, bf0, Wf1, bf1, Wp, bp):
    src = edge_index[0]
    dst = edge_index[1]

    src0 = jnp.full((32, CA * K), 0, jnp.int32).at[:, :10000].set(
        src.reshape(32, 10000)).reshape(32, CA, K)
    dst0 = jnp.full((32, CA * K), GARBAGE, jnp.int32).at[:, :10000].set(
        dst.reshape(32, 10000)).reshape(32, CA, K)
    sd = jnp.stack([src0, dst0], axis=2).reshape(NCHUNK, 2, K)

    z128 = jnp.zeros((NP, 128), jnp.float32)
    ones = jnp.ones((K, 128), jnp.float32)
    x_pad = jnp.zeros((NP, 128), jnp.float32).at[:N].set(x)

    bl0r = bl0.reshape(1, 256)
    bl1r = bl1.reshape(1, 256)
    bf0r = bf0.reshape(1, 256)
    bf1r = bf1.reshape(1, 256)
    bpr = bp.reshape(1, 128)

    agg0p, deg0p = _sc_agg0(x_pad, sd, z128, ones)
    h0st = _tc1(agg0p, deg0p, x_pad, Wl0, bl0r, Wr0)

    h0f = h0st.reshape(2 * NP, 128)
    agg1h = _sc_agg1(h0f, sd, z128)

    out = _tc2(agg1h, deg0p, h0st, Wl1, bl1r, Wr1,
               Wf0, bf0r, Wf1, bf1r, Wp, bpr)
    return out[:N]


# speedup vs baseline: 1.0044x; 1.0044x over previous
import functools

import jax
import jax.numpy as jnp
from jax import lax
from jax.experimental import pallas as pl
from jax.experimental.pallas import tpu as pltpu
from jax.experimental.pallas import tpu_sc as plsc

N = 10000
NP = 10240
E = 320000
GARBAGE = N

K = 128
CA = 80
CB = 160
ROWS_PER_SUB = NP // 16

_mesh = plsc.VectorSubcoreMesh(core_axis_name="c", subcore_axis_name="s")


G = 20
NCHUNK = 32 * CA


def _agg_pipeline(table_hbm, stage_fn, sd_g, rows2, acc_sh, sem_g, sem_s,
                  n_groups):
    def group(g, _):
        stage_fn(g)
        gathers = [None] * G
        scatters = [None] * G
        gathers[0] = pltpu.async_copy(table_hbm.at[sd_g.at[0, 0]],
                                      rows2.at[0], sem_g)
        for j in range(G):
            b = j & 1
            gathers[j].wait()
            if j + 1 < G:
                if j >= 1:
                    scatters[j - 1].wait()
                gathers[j + 1] = pltpu.async_copy(
                    table_hbm.at[sd_g.at[j + 1, 0]], rows2.at[1 - b], sem_g)
            scatters[j] = pltpu.async_copy(
                rows2.at[b], acc_sh.at[sd_g.at[j, 1]], sem_s, add=True)
        scatters[G - 2].wait()
        scatters[G - 1].wait()
        return 0

    lax.fori_loop(0, n_groups, group, 0)


@functools.partial(
    pl.kernel,
    out_type=[
        jax.ShapeDtypeStruct((2, NP, 128), jnp.float32),
        jax.ShapeDtypeStruct((2, NP, 128), jnp.float32),
    ],
    mesh=_mesh,
    scratch_types=[
        pltpu.VMEM((2 * G, 2, K), jnp.int32),
        pltpu.VMEM((2, K, 128), jnp.float32),
        pltpu.VMEM_SHARED((NP, 128), jnp.float32),
        pltpu.SemaphoreType.DMA,
        pltpu.SemaphoreType.DMA,
    ],
)
def _sc_agg0(x_hbm, sd_hbm, z128_hbm, ones_hbm,
             agg_out, deg_out,
             sd_g2, rows2, acc_sh, sem_g, sem_s):
    sd_g = sd_g2.at[pl.ds(0, G)]
    c = lax.axis_index("c")
    s = lax.axis_index("s")
    wid = c * 16 + s
    r0 = s * ROWS_PER_SUB
    pltpu.sync_copy(z128_hbm.at[pl.ds(r0, ROWS_PER_SUB)],
                    acc_sh.at[pl.ds(r0, ROWS_PER_SUB)])
    pltpu.sync_copy(ones_hbm, rows2.at[0])
    plsc.subcore_barrier()

    def deg_group(g, _):
        pltpu.sync_copy(sd_hbm.at[pl.ds(wid * CA + g * (2 * G), 2 * G)], sd_g2)
        scatters = [
            pltpu.async_copy(rows2.at[0], acc_sh.at[sd_g2.at[j, 1]], sem_s,
                             add=True)
            for j in range(2 * G)
        ]
        for sc in scatters:
            sc.wait()
        return 0

    lax.fori_loop(0, CA // (2 * G), deg_group, 0)
    plsc.subcore_barrier()
    pltpu.sync_copy(acc_sh.at[pl.ds(r0, ROWS_PER_SUB)],
                    deg_out.at[c, pl.ds(r0, ROWS_PER_SUB)])
    plsc.subcore_barrier()
    pltpu.sync_copy(z128_hbm.at[pl.ds(r0, ROWS_PER_SUB)],
                    acc_sh.at[pl.ds(r0, ROWS_PER_SUB)])
    plsc.subcore_barrier()

    def stage(g):
        pltpu.sync_copy(sd_hbm.at[pl.ds(wid * CA + g * G, G)], sd_g)

    _agg_pipeline(x_hbm, stage, sd_g, rows2, acc_sh, sem_g, sem_s, CA // G)
    plsc.subcore_barrier()
    pltpu.sync_copy(acc_sh.at[pl.ds(r0, ROWS_PER_SUB)],
                    agg_out.at[c, pl.ds(r0, ROWS_PER_SUB)])


@functools.partial(
    pl.kernel,
    out_type=jax.ShapeDtypeStruct((2, NP, 128), jnp.float32),
    mesh=_mesh,
    scratch_types=[
        pltpu.VMEM((G, 2, K), jnp.int32),
        pltpu.VMEM((2, K, 128), jnp.float32),
        pltpu.VMEM_SHARED((NP, 128), jnp.float32),
        pltpu.SemaphoreType.DMA,
        pltpu.SemaphoreType.DMA,
    ],
)
def _sc_agg1(h_hbm, sd_hbm, z128_hbm,
             agg_out,
             sd_g, rows2, acc_sh, sem_g, sem_s):
    c = lax.axis_index("c")
    s = lax.axis_index("s")
    r0 = s * ROWS_PER_SUB
    pltpu.sync_copy(z128_hbm.at[pl.ds(r0, ROWS_PER_SUB)],
                    acc_sh.at[pl.ds(r0, ROWS_PER_SUB)])
    plsc.subcore_barrier()
    off = c * NP

    def stage(g):
        pltpu.sync_copy(sd_hbm.at[pl.ds(s * CB + g * G, G)], sd_g)
        def add_off(j, _):
            def lane(i, _):
                v = sd_g[j, 0, pl.ds(i * 16, 16)]
                sd_g[j, 0, pl.ds(i * 16, 16)] = v + off
                return 0
            lax.fori_loop(0, K // 16, lane, 0)
            return 0
        lax.fori_loop(0, G, add_off, 0)

    _agg_pipeline(h_hbm, stage, sd_g, rows2, acc_sh, sem_g, sem_s, CB // G)
    plsc.subcore_barrier()
    pltpu.sync_copy(acc_sh.at[pl.ds(r0, ROWS_PER_SUB)],
                    agg_out.at[c, pl.ds(r0, ROWS_PER_SUB)])


def _tc1_body(aggp, degp, x, wl, bl, wr, out):
    a = aggp[0] + aggp[1]
    deg = (degp[0, :, 0].astype(jnp.float32)
           + degp[1, :, 0].astype(jnp.float32))
    r = 1.0 / jnp.maximum(deg, 1.0)
    h = (jnp.dot(a * r[:, None], wl[:], preferred_element_type=jnp.float32)
         + bl[:] + jnp.dot(x[:], wr[:], preferred_element_type=jnp.float32))
    h = jnp.maximum(h, 0.0)
    out[0] = h[:, :128]
    out[1] = h[:, 128:]


def _tc2_body(aggh, degp, h0st, wl, bl, wr, wf0, bf0, wf1, bf1, wp, bp, out):
    agg = jnp.concatenate([aggh[0], aggh[1]], axis=1)
    h0 = jnp.concatenate([h0st[0], h0st[1]], axis=1)
    deg = (degp[0, :, 0].astype(jnp.float32)
           + degp[1, :, 0].astype(jnp.float32))
    r = 1.0 / jnp.maximum(deg, 1.0)
    h = (jnp.dot(agg * r[:, None], wl[:], preferred_element_type=jnp.float32)
         + bl[:] + jnp.dot(h0, wr[:], preferred_element_type=jnp.float32))
    h = jnp.maximum(h, 0.0)
    h = jnp.maximum(jnp.dot(h, wf0[:], preferred_element_type=jnp.float32) + bf0[:], 0.0)
    h = jnp.maximum(jnp.dot(h, wf1[:], preferred_element_type=jnp.float32) + bf1[:], 0.0)
    out[:] = jnp.dot(h, wp[:], preferred_element_type=jnp.float32) + bp[:]


_R = 512
_GRID = NP // _R


def _full(shape):
    return pl.BlockSpec(shape, lambda i: tuple(0 for _ in shape))


_tc1 = pl.pallas_call(
    _tc1_body,
    grid=(_GRID,),
    in_specs=[
        pl.BlockSpec((2, _R, 128), lambda i: (0, i, 0)),
        pl.BlockSpec((2, _R, 128), lambda i: (0, i, 0)),
        pl.BlockSpec((_R, 128), lambda i: (i, 0)),
        _full((128, 256)),
        _full((1, 256)),
        _full((128, 256)),
    ],
    out_specs=pl.BlockSpec((2, _R, 128), lambda i: (0, i, 0)),
    out_shape=jax.ShapeDtypeStruct((2, NP, 128), jnp.float32),
)

_tc2 = pl.pallas_call(
    _tc2_body,
    grid=(_GRID,),
    in_specs=[
        pl.BlockSpec((2, _R, 128), lambda i: (0, i, 0)),
        pl.BlockSpec((2, _R, 128), lambda i: (0, i, 0)),
        pl.BlockSpec((2, _R, 128), lambda i: (0, i, 0)),
        _full((256, 256)),
        _full((1, 256)),
        _full((256, 256)),
        _full((256, 256)),
        _full((1, 256)),
        _full((256, 256)),
        _full((1, 256)),
        _full((256, 128)),
        _full((1, 128)),
    ],
    out_specs=pl.BlockSpec((_R, 128), lambda i: (i, 0)),
    out_shape=jax.ShapeDtypeStruct((NP, 128), jnp.float32),
)


def kernel(x, edge_index, Wl0, bl0, Wr0, Wl1, bl1, Wr1, Wf0, bf0, Wf1, bf1, Wp, bp):
    src = edge_index[0]
    dst = edge_index[1]

    src0 = jnp.full((32, CA * K), 0, jnp.int32).at[:, :10000].set(
        src.reshape(32, 10000)).reshape(32, CA, K)
    dst0 = jnp.full((32, CA * K), GARBAGE, jnp.int32).at[:, :10000].set(
        dst.reshape(32, 10000)).reshape(32, CA, K)
    sd = jnp.stack([src0, dst0], axis=2).reshape(NCHUNK, 2, K)

    z128 = jnp.zeros((NP, 128), jnp.float32)
    ones = jnp.ones((K, 128), jnp.float32)

    bl0r = bl0.reshape(1, 256)
    bl1r = bl1.reshape(1, 256)
    bf0r = bf0.reshape(1, 256)
    bf1r = bf1.reshape(1, 256)
    bpr = bp.reshape(1, 128)

    agg0p, deg0p = _sc_agg0(x, sd, z128, ones)
    h0st = _tc1(agg0p, deg0p, x, Wl0, bl0r, Wr0)

    h0f = h0st.reshape(2 * NP, 128)
    agg1h = _sc_agg1(h0f, sd, z128)

    out = _tc2(agg1h, deg0p, h0st, Wl1, bl1r, Wr1,
               Wf0, bf0r, Wf1, bf1r, Wp, bpr)
    return out[:N]
